# baseline (speedup 1.0000x reference)
import functools

import jax
import jax.numpy as jnp
from jax import lax
from jax.experimental import pallas as pl
from jax.experimental.pallas import tpu as pltpu
from jax.experimental.pallas import tpu_sc as plsc

_F = 32
_L = 16


def _mf_body(user_hbm, item_hbm, uflat_hbm, iflat_hbm, out_hbm,
             uidx_v, iidx_v, uoff_v, ioff_v, uvals_v, ivals_v, out_v,
             usem, isem, *, b_per_w, num_cores, n_rows):
    wid = lax.axis_index("s") * num_cores + lax.axis_index("c")
    base = wid * b_per_w

    pltpu.sync_copy(user_hbm.at[pl.ds(base, b_per_w)], uidx_v)
    pltpu.sync_copy(item_hbm.at[pl.ds(base, b_per_w)], iidx_v)

    n_chunks = b_per_w // _L

    def offsets(c, carry):
        ui = uidx_v[pl.ds(c * _L, _L)]
        ii = iidx_v[pl.ds(c * _L, _L)]
        for f in range(_F):
            uoff_v[pl.ds(f * b_per_w + c * _L, _L)] = ui + f * n_rows
            ioff_v[pl.ds(f * b_per_w + c * _L, _L)] = ii + f * n_rows
        return carry

    lax.fori_loop(0, n_chunks, offsets, 0, unroll=False)

    ucp = pltpu.async_copy(uflat_hbm.at[uoff_v], uvals_v, usem)
    icp = pltpu.async_copy(iflat_hbm.at[ioff_v], ivals_v, isem)
    ucp.wait()
    icp.wait()

    def reduce_chunk(c, carry):
        acc = jnp.zeros((_L,), jnp.float32)
        for f in range(_F):
            u = uvals_v[pl.ds(f * b_per_w + c * _L, _L)]
            v = ivals_v[pl.ds(f * b_per_w + c * _L, _L)]
            acc = acc + u * v
        out_v[pl.ds(c * _L, _L)] = acc
        return carry

    lax.fori_loop(0, n_chunks, reduce_chunk, 0, unroll=False)

    pltpu.sync_copy(out_v, out_hbm.at[pl.ds(base, b_per_w)])


def kernel(user, item, user_factors, item_factors):
    batch = user.shape[0]
    n_rows, n_factors = user_factors.shape
    assert n_factors == _F

    info = plsc.get_sparse_core_info()
    nw = info.num_cores * info.num_subcores
    b_per_w = batch // nw
    assert b_per_w * nw == batch and b_per_w % _L == 0

    mesh = plsc.VectorSubcoreMesh(core_axis_name="c", subcore_axis_name="s")

    mf = pl.kernel(
        functools.partial(_mf_body, b_per_w=b_per_w,
                          num_cores=info.num_cores, n_rows=n_rows),
        out_type=jax.ShapeDtypeStruct((batch,), jnp.float32),
        mesh=mesh,
        compiler_params=pltpu.CompilerParams(
            needs_layout_passes=False, use_tc_tiling_on_sc=False),
        scratch_types=[
            pltpu.VMEM((b_per_w,), jnp.int32),
            pltpu.VMEM((b_per_w,), jnp.int32),
            pltpu.VMEM((_F * b_per_w,), jnp.int32),
            pltpu.VMEM((_F * b_per_w,), jnp.int32),
            pltpu.VMEM((_F * b_per_w,), jnp.float32),
            pltpu.VMEM((_F * b_per_w,), jnp.float32),
            pltpu.VMEM((b_per_w,), jnp.float32),
            pltpu.SemaphoreType.DMA,
            pltpu.SemaphoreType.DMA,
        ],
    )
    return mf(user.astype(jnp.int32), item.astype(jnp.int32),
              user_factors.T.reshape(-1), item_factors.T.reshape(-1))
